# baseline (speedup 1.0000x reference)
import math

import jax
import jax.numpy as jnp
from jax import lax
from jax.experimental import pallas as pl
from jax.experimental.pallas import tpu as pltpu

_BLOCK_ROWS = 256
_SUB = 64
_HALF_PI = math.pi / 2.0


def _make_gen_block(hidden):
    c = 2.0 * math.log(10000.0) / hidden
    chunks = _BLOCK_ROWS // _SUB

    def _gen_block(o_ref, sinb_ref, cosb_ref, sina_ref, cosa_ref, stp_ref):
        pid = pl.program_id(0)

        @pl.when(pid == 0)
        def _init():
            icol = jax.lax.broadcasted_iota(jnp.int32, (1, hidden), 1)
            f = jnp.exp(icol.astype(jnp.float32) * (-c))
            phase = (icol & 1).astype(jnp.float32) * _HALF_PI
            d8 = jax.lax.broadcasted_iota(jnp.int32, (8, 1), 0)
            angd = d8.astype(jnp.float32) * f + phase
            sind = jnp.sin(angd)
            cosd = jnp.sin(angd + _HALF_PI)
            angc = (d8 * 8).astype(jnp.float32) * f
            sinc = jnp.sin(angc)
            cosc = jnp.sin(angc + _HALF_PI)
            for cc in range(8):
                sc = lax.slice(sinc, (cc, 0), (cc + 1, hidden))
                kc = lax.slice(cosc, (cc, 0), (cc + 1, hidden))
                sinb_ref[pl.ds(cc * 8, 8), :] = sc * cosd + kc * sind
                cosb_ref[pl.ds(cc * 8, 8), :] = kc * cosd - sc * sind
            anga = (d8 * _SUB).astype(jnp.float32) * f
            sina_ref[...] = jnp.sin(anga)
            cosa_ref[...] = jnp.sin(anga + _HALF_PI)
            angs = jnp.float32(_BLOCK_ROWS) * f
            stp_ref[0:1, :] = jnp.sin(angs)
            stp_ref[1:2, :] = jnp.sin(angs + _HALF_PI)

        sinb = sinb_ref[...]
        cosb = cosb_ref[...]
        sina_blk = sina_ref[...]
        cosa_blk = cosa_ref[...]
        for k in range(chunks):
            sina = lax.slice(sina_blk, (k, 0), (k + 1, hidden))
            cosa = lax.slice(cosa_blk, (k, 0), (k + 1, hidden))
            o_ref[pl.ds(k * _SUB, _SUB), :] = sina * cosb + cosa * sinb

        @pl.when(pid == 0)
        def _zero_row0():
            o_ref[0:1, :] = jnp.zeros((1, hidden), jnp.float32)

        sstp = stp_ref[0:1, :]
        cstp = stp_ref[1:2, :]
        sina_ref[...] = sina_blk * cstp + cosa_blk * sstp
        cosa_ref[...] = cosa_blk * cstp - sina_blk * sstp

    return _gen_block


def kernel(x, table):
    seq_len = x.shape[-1]
    hidden = table.shape[1]
    return pl.pallas_call(
        _make_gen_block(hidden),
        grid=(seq_len // _BLOCK_ROWS,),
        out_specs=pl.BlockSpec((_BLOCK_ROWS, hidden), lambda i: (i, 0)),
        out_shape=jax.ShapeDtypeStruct((seq_len, hidden), table.dtype),
        scratch_shapes=[
            pltpu.VMEM((_SUB, hidden), jnp.float32),
            pltpu.VMEM((_SUB, hidden), jnp.float32),
            pltpu.VMEM((8, hidden), jnp.float32),
            pltpu.VMEM((8, hidden), jnp.float32),
            pltpu.VMEM((2, hidden), jnp.float32),
        ],
    )()


# speedup vs baseline: 1.3739x; 1.3739x over previous
import math

import jax
import jax.numpy as jnp
from jax import lax
from jax.experimental import pallas as pl
from jax.experimental.pallas import tpu as pltpu

_BLOCK_ROWS = 1024
_SUB = 64
_HALF_PI = math.pi / 2.0


def _make_gen_block(hidden):
    c = 2.0 * math.log(10000.0) / hidden
    chunks = _BLOCK_ROWS // _SUB

    def _gen_block(o_ref, sinb_ref, cosb_ref, sina_ref, cosa_ref, stp_ref):
        pid = pl.program_id(0)

        @pl.when(pid == 0)
        def _init():
            icol = jax.lax.broadcasted_iota(jnp.int32, (1, hidden), 1)
            f = jnp.exp(icol.astype(jnp.float32) * (-c))
            phase = (icol & 1).astype(jnp.float32) * _HALF_PI
            d8 = jax.lax.broadcasted_iota(jnp.int32, (8, 1), 0)
            angd = d8.astype(jnp.float32) * f + phase
            sind = jnp.sin(angd)
            cosd = jnp.sin(angd + _HALF_PI)
            angc = (d8 * 8).astype(jnp.float32) * f
            sinc = jnp.sin(angc)
            cosc = jnp.sin(angc + _HALF_PI)
            for cc in range(8):
                sc = lax.slice(sinc, (cc, 0), (cc + 1, hidden))
                kc = lax.slice(cosc, (cc, 0), (cc + 1, hidden))
                sinb_ref[pl.ds(cc * 8, 8), :] = sc * cosd + kc * sind
                cosb_ref[pl.ds(cc * 8, 8), :] = kc * cosd - sc * sind
            ka = jax.lax.broadcasted_iota(jnp.int32, (chunks, 1), 0)
            anga = (ka * _SUB).astype(jnp.float32) * f
            sina_ref[...] = jnp.sin(anga)
            cosa_ref[...] = jnp.sin(anga + _HALF_PI)
            angs = jnp.float32(_BLOCK_ROWS) * f
            stp_ref[0:1, :] = jnp.sin(angs)
            stp_ref[1:2, :] = jnp.sin(angs + _HALF_PI)

        sinb = sinb_ref[...]
        cosb = cosb_ref[...]
        sina_blk = sina_ref[...]
        cosa_blk = cosa_ref[...]
        for k in range(chunks):
            sina = lax.slice(sina_blk, (k, 0), (k + 1, hidden))
            cosa = lax.slice(cosa_blk, (k, 0), (k + 1, hidden))
            o_ref[pl.ds(k * _SUB, _SUB), :] = sina * cosb + cosa * sinb

        @pl.when(pid == 0)
        def _zero_row0():
            o_ref[0:1, :] = jnp.zeros((1, hidden), jnp.float32)

        sstp = stp_ref[0:1, :]
        cstp = stp_ref[1:2, :]
        sina_ref[...] = sina_blk * cstp + cosa_blk * sstp
        cosa_ref[...] = cosa_blk * cstp - sina_blk * sstp

    return _gen_block


def kernel(x, table):
    seq_len = x.shape[-1]
    hidden = table.shape[1]
    return pl.pallas_call(
        _make_gen_block(hidden),
        grid=(seq_len // _BLOCK_ROWS,),
        out_specs=pl.BlockSpec((_BLOCK_ROWS, hidden), lambda i: (i, 0)),
        out_shape=jax.ShapeDtypeStruct((seq_len, hidden), table.dtype),
        scratch_shapes=[
            pltpu.VMEM((_SUB, hidden), jnp.float32),
            pltpu.VMEM((_SUB, hidden), jnp.float32),
            pltpu.VMEM((_BLOCK_ROWS // _SUB, hidden), jnp.float32),
            pltpu.VMEM((_BLOCK_ROWS // _SUB, hidden), jnp.float32),
            pltpu.VMEM((2, hidden), jnp.float32),
        ],
    )()


# speedup vs baseline: 1.5523x; 1.1298x over previous
import math

import jax
import jax.numpy as jnp
from jax import lax
from jax.experimental import pallas as pl
from jax.experimental.pallas import tpu as pltpu

_CHUNK_ROWS = 256
_SUB = 64
_NBUF = 4
_HALF_PI = math.pi / 2.0


def _make_body(seq_len, hidden):
    c = 2.0 * math.log(10000.0) / hidden
    sub_per_chunk = _CHUNK_ROWS // _SUB
    nchunks = seq_len // _CHUNK_ROWS

    def _body(o_ref, buf, *sems):
        icol = jax.lax.broadcasted_iota(jnp.int32, (1, hidden), 1)
        f = jnp.exp(icol.astype(jnp.float32) * (-c))
        phase = (icol & 1).astype(jnp.float32) * _HALF_PI

        d8 = jax.lax.broadcasted_iota(jnp.int32, (8, 1), 0)
        angd = d8.astype(jnp.float32) * f + phase
        sind = jnp.sin(angd)
        cosd = jnp.sin(angd + _HALF_PI)
        angc = (d8 * 8).astype(jnp.float32) * f
        sinc = jnp.sin(angc)
        cosc = jnp.sin(angc + _HALF_PI)
        sinb_rows = []
        cosb_rows = []
        for cc in range(8):
            sc = lax.slice(sinc, (cc, 0), (cc + 1, hidden))
            kc = lax.slice(cosc, (cc, 0), (cc + 1, hidden))
            sinb_rows.append(sc * cosd + kc * sind)
            cosb_rows.append(kc * cosd - sc * sind)
        sinb = lax.concatenate(sinb_rows, 0)
        cosb = lax.concatenate(cosb_rows, 0)

        ka = jax.lax.broadcasted_iota(jnp.int32, (sub_per_chunk, 1), 0)
        anga = (ka * _SUB).astype(jnp.float32) * f
        sina_t = jnp.sin(anga)
        cosa_t = jnp.sin(anga + _HALF_PI)
        angs = jnp.float32(_CHUNK_ROWS) * f
        sstp = jnp.sin(angs)
        cstp = jnp.sin(angs + _HALF_PI)

        stores = [None] * _NBUF
        for j in range(nchunks):
            bsel = j % _NBUF
            if stores[bsel] is not None:
                stores[bsel].wait()
            for kk in range(sub_per_chunk):
                sina = lax.slice(sina_t, (kk, 0), (kk + 1, hidden))
                cosa = lax.slice(cosa_t, (kk, 0), (kk + 1, hidden))
                val = sina * cosb + cosa * sinb
                if j == 0 and kk == 0:
                    irow = jax.lax.broadcasted_iota(
                        jnp.int32, (_SUB, 1), 0
                    )
                    val = jnp.where(irow == 0, 0.0, val)
                buf[bsel, pl.ds(kk * _SUB, _SUB), :] = val
            stores[bsel] = pltpu.make_async_copy(
                buf.at[bsel],
                o_ref.at[pl.ds(j * _CHUNK_ROWS, _CHUNK_ROWS)],
                sems[bsel],
            )
            stores[bsel].start()
            new_sina = sina_t * cstp + cosa_t * sstp
            cosa_t = cosa_t * cstp - sina_t * sstp
            sina_t = new_sina
        for h in stores:
            if h is not None:
                h.wait()

    return _body


def kernel(x, table):
    seq_len = x.shape[-1]
    hidden = table.shape[1]
    return pl.pallas_call(
        _make_body(seq_len, hidden),
        out_specs=pl.BlockSpec(memory_space=pl.ANY),
        out_shape=jax.ShapeDtypeStruct((seq_len, hidden), table.dtype),
        scratch_shapes=[pltpu.VMEM((_NBUF, _CHUNK_ROWS, hidden), table.dtype)]
        + [pltpu.SemaphoreType.DMA] * _NBUF,
    )()
